# baseline (speedup 1.0000x reference)
import functools

import jax
import jax.numpy as jnp
from jax import lax
from jax.experimental import pallas as pl
from jax.experimental.pallas import tpu as pltpu
from jax.experimental.pallas import tpu_sc as plsc

_N = 100000
_E = 3200000
_NP = 102400
_NA = 100352
_CH = 80
_K = 10
_MACRO = _CH * _K
_NTILES = 16
_NW = 32

_B = 2048
_G = (_N + _B - 1) // _B


def _sc_mesh():
    return plsc.VectorSubcoreMesh(core_axis_name="c", subcore_axis_name="s")


@functools.partial(
    pl.kernel,
    out_type=jax.ShapeDtypeStruct((2, _NA, 16), jnp.float32),
    mesh=_sc_mesh(),
    compiler_params=pltpu.CompilerParams(use_tc_tiling_on_sc=False),
    scratch_types=[
        pltpu.VMEM_SHARED((_NA, 16), jnp.float32),
        pltpu.VMEM((2, _MACRO), jnp.int32),
        pltpu.VMEM((_MACRO, 16), jnp.float32),
        pltpu.SemaphoreType.DMA,
        pltpu.SemaphoreType.DMA,
    ],
)
def _deg_kernel(row3_hbm, ones_hbm, zeros_hbm, out_hbm, acc, ridx, ones,
                isem, ssem):
    c = lax.axis_index("c")
    s = lax.axis_index("s")
    wid = c * _NTILES + s
    pltpu.sync_copy(ones_hbm, ones)
    zr = _NA // _NTILES
    pltpu.sync_copy(zeros_hbm, acc.at[pl.ds(s * zr, zr)])
    plsc.subcore_barrier()

    nm = _E // _MACRO // _NW
    base_m = wid * nm

    pltpu.async_copy(row3_hbm.at[base_m], ridx.at[0], isem)

    @pl.loop(0, nm)
    def _loop(t):
        p = lax.rem(t, 2)
        q = 1 - p
        pltpu.make_async_copy(row3_hbm.at[base_m], ridx.at[p], isem).wait()

        @pl.when(t >= 1)
        def _():
            pltpu.make_async_copy(ones, acc.at[ridx.at[p]], ssem).wait()

        @pl.when(t + 1 < nm)
        def _():
            pltpu.async_copy(row3_hbm.at[base_m + t + 1], ridx.at[q], isem)

        pltpu.async_copy(ones, acc.at[ridx.at[p]], ssem, add=True)

    pltpu.make_async_copy(ones, acc.at[ridx.at[0]], ssem).wait()
    plsc.subcore_barrier()
    pltpu.sync_copy(acc.at[pl.ds(s * zr, zr)], out_hbm.at[c, pl.ds(s * zr, zr)])


def _make_agg(edge_split):
    nm = (_E // _MACRO) // (_NW if edge_split else _NTILES)
    table_shape = (_NA, 16) if edge_split else (2, _NA, 16)

    @functools.partial(
        pl.kernel,
        out_type=jax.ShapeDtypeStruct((2, _NA, 16), jnp.float32),
        mesh=_sc_mesh(),
        compiler_params=pltpu.CompilerParams(use_tc_tiling_on_sc=False),
        scratch_types=[
            pltpu.VMEM_SHARED((_NA, 16), jnp.float32),
            pltpu.VMEM((2, _MACRO), jnp.int32),
            pltpu.VMEM((2, _MACRO), jnp.int32),
            pltpu.VMEM((2, _MACRO, 16), jnp.float32),
            pltpu.SemaphoreType.DMA,
            pltpu.SemaphoreType.DMA,
            pltpu.SemaphoreType.DMA,
        ],
    )
    def agg(table_hbm, row3_hbm, col3_hbm, zeros_hbm, out_hbm,
            acc, ridx, cidx, rows, isem, gsem, ssem):
        c = lax.axis_index("c")
        s = lax.axis_index("s")
        base_m = (c * _NTILES + s if edge_split else s) * nm
        tbl = table_hbm if edge_split else table_hbm.at[c]

        zr = _NA // _NTILES
        pltpu.sync_copy(zeros_hbm, acc.at[pl.ds(s * zr, zr)])
        plsc.subcore_barrier()

        pltpu.async_copy(row3_hbm.at[base_m], ridx.at[0], isem)
        pltpu.async_copy(col3_hbm.at[base_m], cidx.at[0], isem)

        @pl.loop(0, nm)
        def _loop(t):
            p = lax.rem(t, 2)
            q = 1 - p
            pltpu.make_async_copy(row3_hbm.at[base_m], ridx.at[p], isem).wait()
            pltpu.make_async_copy(col3_hbm.at[base_m], cidx.at[p], isem).wait()
            gc = pltpu.async_copy(tbl.at[ridx.at[p]], rows.at[p], gsem)

            @pl.when(t >= 1)
            def _():
                pltpu.make_async_copy(zeros_hbm, rows.at[p], ssem).wait()

            @pl.when(t + 1 < nm)
            def _():
                pltpu.async_copy(row3_hbm.at[base_m + t + 1], ridx.at[q], isem)
                pltpu.async_copy(col3_hbm.at[base_m + t + 1], cidx.at[q], isem)

            gc.wait()
            pltpu.async_copy(rows.at[p], acc.at[cidx.at[p]], ssem, add=True)

        pltpu.make_async_copy(zeros_hbm, rows.at[0], ssem).wait()

        plsc.subcore_barrier()
        pltpu.sync_copy(acc.at[pl.ds(s * zr, zr)],
                        out_hbm.at[c, pl.ds(s * zr, zr)])

    return agg


_agg_edge_split = _make_agg(True)
_agg_feat_split = _make_agg(False)


_B8 = _B // 8
_N8 = _NA // 8


def _tc1_body(x_ref, w_ref, deg_ref, y1_ref):
    deg = deg_ref[0] + deg_ref[1] + 1.0
    dis = lax.rsqrt(deg)
    h = jnp.dot(x_ref[:, 0, :], w_ref[0],
                preferred_element_type=jnp.float32)
    for i in range(1, 8):
        h += jnp.dot(x_ref[:, i, :], w_ref[i],
                     preferred_element_type=jnp.float32)
    y1_ref[...] = h * dis


def _tc1(x3, W1big, deg_v):
    return pl.pallas_call(
        _tc1_body,
        grid=(_G,),
        in_specs=[
            pl.BlockSpec((_B8, 8, 128), lambda i: (i, 0, 0)),
            pl.BlockSpec((8, 128, 128), lambda i: (0, 0, 0)),
            pl.BlockSpec((2, _B8, 128), lambda i: (0, i, 0)),
        ],
        out_specs=pl.BlockSpec((_B8, 128), lambda i: (i, 0)),
        out_shape=jax.ShapeDtypeStruct((_N8, 128), jnp.float32),
    )(x3, W1big, deg_v)


def _tc2_body(ag_ref, y1_ref, deg_ref, b1_ref, w2_ref, y2_ref):
    deg = deg_ref[0] + deg_ref[1] + 1.0
    dis = lax.rsqrt(deg)
    h = jnp.maximum(dis * (ag_ref[0] + ag_ref[1] + y1_ref[...]) + b1_ref[...],
                    0.0)
    y2_ref[0] = dis * jnp.dot(h, w2_ref[0],
                              preferred_element_type=jnp.float32)
    y2_ref[1] = dis * jnp.dot(h, w2_ref[1],
                              preferred_element_type=jnp.float32)


def _tc2(agg1_v, y1_pk, deg_v, b1_p, W2big):
    return pl.pallas_call(
        _tc2_body,
        grid=(_G,),
        in_specs=[
            pl.BlockSpec((2, _B8, 128), lambda i: (0, i, 0)),
            pl.BlockSpec((_B8, 128), lambda i: (i, 0)),
            pl.BlockSpec((2, _B8, 128), lambda i: (0, i, 0)),
            pl.BlockSpec((1, 128), lambda i: (0, 0)),
            pl.BlockSpec((2, 128, 128), lambda i: (0, 0, 0)),
        ],
        out_specs=pl.BlockSpec((2, _B8, 128), lambda i: (0, i, 0)),
        out_shape=jax.ShapeDtypeStruct((2, _N8, 128), jnp.float32),
    )(agg1_v, y1_pk, deg_v, b1_p, W2big)


def _tc3_body(ag_ref, y2_ref, deg_ref, b2_ref, s_ref, oa_ref, ob_ref):
    deg = deg_ref[0] + deg_ref[1] + 1.0
    dis = lax.rsqrt(deg)
    za = dis * (ag_ref[0] + y2_ref[0]) + b2_ref[0]
    zb = dis * (ag_ref[1] + y2_ref[1]) + b2_ref[1]
    m8 = jnp.max(jnp.maximum(za, zb), axis=1, keepdims=True)
    ea = jnp.exp(za - m8)
    eb = jnp.exp(zb - m8)
    se = jnp.dot(ea + eb, s_ref[...], preferred_element_type=jnp.float32)
    lse = m8 + jnp.log(se)
    oa_ref[...] = za - lse
    ob_ref[...] = zb - lse


def _tc3(agg2_v, y2_pk, deg_v, b2_p, S):
    return pl.pallas_call(
        _tc3_body,
        grid=(_G,),
        in_specs=[
            pl.BlockSpec((2, _B8, 128), lambda i: (0, i, 0)),
            pl.BlockSpec((2, _B8, 128), lambda i: (0, i, 0)),
            pl.BlockSpec((2, _B8, 128), lambda i: (0, i, 0)),
            pl.BlockSpec((2, 1, 128), lambda i: (0, 0, 0)),
            pl.BlockSpec((128, 128), lambda i: (0, 0)),
        ],
        out_specs=[
            pl.BlockSpec((_B8, 128), lambda i: (i, 0)),
            pl.BlockSpec((_B8, 128), lambda i: (i, 0)),
        ],
        out_shape=[
            jax.ShapeDtypeStruct((_N8, 128), jnp.float32),
            jax.ShapeDtypeStruct((_N8, 128), jnp.float32),
        ],
    )(agg2_v, y2_pk, deg_v, b2_p, S)


def kernel(x, edge_index, W1, b1, W2, b2):
    row3 = edge_index[0].astype(jnp.int32).reshape(-1, _MACRO)
    col3 = edge_index[1].astype(jnp.int32).reshape(-1, _MACRO)
    ones16 = jnp.ones((_MACRO, 16), jnp.float32)
    zeros_n16 = jnp.zeros((_NA // _NTILES, 16), jnp.float32)

    eye8 = jnp.eye(8, dtype=jnp.float32)
    W1big = jnp.kron(eye8, W1).reshape(8, 128, 128)
    W2big = jnp.stack([jnp.kron(eye8, W2[:, :16]),
                       jnp.kron(eye8, W2[:, 16:])])
    S = jnp.kron(eye8, jnp.ones((16, 16), jnp.float32))
    b1_p = jnp.tile(b1, 8)[None]
    b2_p = jnp.stack([jnp.tile(b2[:16], 8)[None],
                      jnp.tile(b2[16:], 8)[None]])

    x3 = x.reshape(-1, 8, 128)

    deg = _deg_kernel(row3, ones16, zeros_n16)
    deg_v = deg.reshape(2, _N8, 128)
    y1_pk = _tc1(x3, W1big, deg_v)
    agg1 = _agg_edge_split(y1_pk.reshape(_NA, 16), row3, col3, zeros_n16)
    y2_pk = _tc2(agg1.reshape(2, _N8, 128), y1_pk, deg_v, b1_p, W2big)
    agg2 = _agg_feat_split(y2_pk.reshape(2, _NA, 16), row3, col3, zeros_n16)
    oa, ob = _tc3(agg2.reshape(2, _N8, 128), y2_pk, deg_v, b2_p, S)
    return jnp.concatenate([oa.reshape(_NA, 16)[:_N],
                            ob.reshape(_NA, 16)[:_N]], axis=1)


# speedup vs baseline: 1.0169x; 1.0169x over previous
import functools

import jax
import jax.numpy as jnp
from jax import lax
from jax.experimental import pallas as pl
from jax.experimental.pallas import tpu as pltpu
from jax.experimental.pallas import tpu_sc as plsc

_N = 100000
_E = 3200000
_NP = 102400
_NA = 100352
_CH = 80
_K = 10
_MACRO = _CH * _K
_NTILES = 16
_NW = 32

_B = 2048
_G = (_N + _B - 1) // _B


def _sc_mesh():
    return plsc.VectorSubcoreMesh(core_axis_name="c", subcore_axis_name="s")


@functools.partial(
    pl.kernel,
    out_type=jax.ShapeDtypeStruct((2, _NA), jnp.float32),
    mesh=_sc_mesh(),
    compiler_params=pltpu.CompilerParams(use_tc_tiling_on_sc=False),
    scratch_types=[
        pltpu.VMEM_SHARED((_NA,), jnp.float32),
        pltpu.VMEM((2, _MACRO), jnp.int32),
        pltpu.VMEM((_MACRO,), jnp.float32),
        pltpu.SemaphoreType.DMA,
        pltpu.SemaphoreType.DMA,
    ],
)
def _deg_kernel(row3_hbm, ones_hbm, zeros_hbm, out_hbm, acc, ridx, ones,
                isem, ssem):
    c = lax.axis_index("c")
    s = lax.axis_index("s")
    wid = c * _NTILES + s
    pltpu.sync_copy(ones_hbm, ones)
    zr = _NA // _NTILES
    pltpu.sync_copy(zeros_hbm, acc.at[pl.ds(s * zr, zr)])
    plsc.subcore_barrier()

    nm = _E // _MACRO // _NW
    base_m = wid * nm

    pltpu.async_copy(row3_hbm.at[base_m], ridx.at[0], isem)

    @pl.loop(0, nm)
    def _loop(t):
        p = lax.rem(t, 2)
        q = 1 - p
        pltpu.make_async_copy(row3_hbm.at[base_m], ridx.at[p], isem).wait()

        @pl.when(t >= 1)
        def _():
            pltpu.make_async_copy(ones, acc.at[ridx.at[p]], ssem).wait()

        @pl.when(t + 1 < nm)
        def _():
            pltpu.async_copy(row3_hbm.at[base_m + t + 1], ridx.at[q], isem)

        pltpu.async_copy(ones, acc.at[ridx.at[p]], ssem, add=True)

    pltpu.make_async_copy(ones, acc.at[ridx.at[0]], ssem).wait()
    plsc.subcore_barrier()
    pltpu.sync_copy(acc.at[pl.ds(s * zr, zr)], out_hbm.at[c, pl.ds(s * zr, zr)])


def _make_agg(edge_split):
    nm = (_E // _MACRO) // (_NW if edge_split else _NTILES)
    table_shape = (_NA, 16) if edge_split else (2, _NA, 16)

    @functools.partial(
        pl.kernel,
        out_type=jax.ShapeDtypeStruct((2, _NA, 16), jnp.float32),
        mesh=_sc_mesh(),
        compiler_params=pltpu.CompilerParams(use_tc_tiling_on_sc=False),
        scratch_types=[
            pltpu.VMEM_SHARED((_NA, 16), jnp.float32),
            pltpu.VMEM((2, _MACRO), jnp.int32),
            pltpu.VMEM((2, _MACRO), jnp.int32),
            pltpu.VMEM((2, _MACRO, 16), jnp.float32),
            pltpu.SemaphoreType.DMA,
            pltpu.SemaphoreType.DMA,
            pltpu.SemaphoreType.DMA,
        ],
    )
    def agg(table_hbm, row3_hbm, col3_hbm, zeros_hbm, out_hbm,
            acc, ridx, cidx, rows, isem, gsem, ssem):
        c = lax.axis_index("c")
        s = lax.axis_index("s")
        base_m = (c * _NTILES + s if edge_split else s) * nm
        tbl = table_hbm if edge_split else table_hbm.at[c]

        zr = _NA // _NTILES
        pltpu.sync_copy(zeros_hbm, acc.at[pl.ds(s * zr, zr)])
        plsc.subcore_barrier()

        pltpu.async_copy(row3_hbm.at[base_m], ridx.at[0], isem)
        pltpu.async_copy(col3_hbm.at[base_m], cidx.at[0], isem)

        @pl.loop(0, nm)
        def _loop(t):
            p = lax.rem(t, 2)
            q = 1 - p
            pltpu.make_async_copy(row3_hbm.at[base_m], ridx.at[p], isem).wait()
            pltpu.make_async_copy(col3_hbm.at[base_m], cidx.at[p], isem).wait()
            gc = pltpu.async_copy(tbl.at[ridx.at[p]], rows.at[p], gsem)

            @pl.when(t >= 1)
            def _():
                pltpu.make_async_copy(zeros_hbm, rows.at[p], ssem).wait()

            @pl.when(t + 1 < nm)
            def _():
                pltpu.async_copy(row3_hbm.at[base_m + t + 1], ridx.at[q], isem)
                pltpu.async_copy(col3_hbm.at[base_m + t + 1], cidx.at[q], isem)

            gc.wait()
            pltpu.async_copy(rows.at[p], acc.at[cidx.at[p]], ssem, add=True)

        pltpu.make_async_copy(zeros_hbm, rows.at[0], ssem).wait()

        plsc.subcore_barrier()
        pltpu.sync_copy(acc.at[pl.ds(s * zr, zr)],
                        out_hbm.at[c, pl.ds(s * zr, zr)])

    return agg


_agg_edge_split = _make_agg(True)
_agg_feat_split = _make_agg(False)


_B8 = _B // 8
_N8 = _NA // 8


def _dis_pk(deg_ref, r_ref):
    d8 = deg_ref[0] + deg_ref[1] + 1.0
    return jnp.dot(lax.rsqrt(d8), r_ref[...],
                   preferred_element_type=jnp.float32)


def _tc1a_body(x_ref, w_ref, h1_ref):
    h = jnp.dot(x_ref[:, 0, :], w_ref[0],
                preferred_element_type=jnp.float32)
    for i in range(1, 8):
        h += jnp.dot(x_ref[:, i, :], w_ref[i],
                     preferred_element_type=jnp.float32)
    h1_ref[...] = h


def _tc1a(x3, W1big):
    return pl.pallas_call(
        _tc1a_body,
        grid=(_G,),
        in_specs=[
            pl.BlockSpec((_B8, 8, 128), lambda i: (i, 0, 0)),
            pl.BlockSpec((8, 128, 128), lambda i: (0, 0, 0)),
        ],
        out_specs=pl.BlockSpec((_B8, 128), lambda i: (i, 0)),
        out_shape=jax.ShapeDtypeStruct((_N8, 128), jnp.float32),
    )(x3, W1big)


def _tc1b_body(h1_ref, deg_ref, r_ref, y1_ref):
    y1_ref[...] = h1_ref[...] * _dis_pk(deg_ref, r_ref)


def _tc1b(h1_pk, deg_v, R):
    return pl.pallas_call(
        _tc1b_body,
        grid=(_G,),
        in_specs=[
            pl.BlockSpec((_B8, 128), lambda i: (i, 0)),
            pl.BlockSpec((2, _B8, 8), lambda i: (0, i, 0)),
            pl.BlockSpec((8, 128), lambda i: (0, 0)),
        ],
        out_specs=pl.BlockSpec((_B8, 128), lambda i: (i, 0)),
        out_shape=jax.ShapeDtypeStruct((_N8, 128), jnp.float32),
    )(h1_pk, deg_v, R)


def _tc2_body(ag_ref, y1_ref, deg_ref, r_ref, b1_ref, w2_ref, y2_ref):
    dis = _dis_pk(deg_ref, r_ref)
    h = jnp.maximum(dis * (ag_ref[0] + ag_ref[1] + y1_ref[...]) + b1_ref[...],
                    0.0)
    y2_ref[0] = dis * jnp.dot(h, w2_ref[0],
                              preferred_element_type=jnp.float32)
    y2_ref[1] = dis * jnp.dot(h, w2_ref[1],
                              preferred_element_type=jnp.float32)


def _tc2(agg1_v, y1_pk, deg_v, R, b1_p, W2big):
    return pl.pallas_call(
        _tc2_body,
        grid=(_G,),
        in_specs=[
            pl.BlockSpec((2, _B8, 128), lambda i: (0, i, 0)),
            pl.BlockSpec((_B8, 128), lambda i: (i, 0)),
            pl.BlockSpec((2, _B8, 8), lambda i: (0, i, 0)),
            pl.BlockSpec((8, 128), lambda i: (0, 0)),
            pl.BlockSpec((1, 128), lambda i: (0, 0)),
            pl.BlockSpec((2, 128, 128), lambda i: (0, 0, 0)),
        ],
        out_specs=pl.BlockSpec((2, _B8, 128), lambda i: (0, i, 0)),
        out_shape=jax.ShapeDtypeStruct((2, _N8, 128), jnp.float32),
    )(agg1_v, y1_pk, deg_v, R, b1_p, W2big)


def _tc3_body(ag_ref, y2_ref, deg_ref, r_ref, b2_ref, s_ref, oa_ref, ob_ref):
    dis = _dis_pk(deg_ref, r_ref)
    za = dis * (ag_ref[0] + y2_ref[0]) + b2_ref[0]
    zb = dis * (ag_ref[1] + y2_ref[1]) + b2_ref[1]
    m8 = jnp.max(jnp.maximum(za, zb), axis=1, keepdims=True)
    ea = jnp.exp(za - m8)
    eb = jnp.exp(zb - m8)
    se = jnp.dot(ea + eb, s_ref[...], preferred_element_type=jnp.float32)
    lse = m8 + jnp.log(se)
    oa_ref[...] = za - lse
    ob_ref[...] = zb - lse


def _tc3(agg2_v, y2_pk, deg_v, R, b2_p, S):
    return pl.pallas_call(
        _tc3_body,
        grid=(_G,),
        in_specs=[
            pl.BlockSpec((2, _B8, 128), lambda i: (0, i, 0)),
            pl.BlockSpec((2, _B8, 128), lambda i: (0, i, 0)),
            pl.BlockSpec((2, _B8, 8), lambda i: (0, i, 0)),
            pl.BlockSpec((8, 128), lambda i: (0, 0)),
            pl.BlockSpec((2, 1, 128), lambda i: (0, 0, 0)),
            pl.BlockSpec((128, 128), lambda i: (0, 0)),
        ],
        out_specs=[
            pl.BlockSpec((_B8, 128), lambda i: (i, 0)),
            pl.BlockSpec((_B8, 128), lambda i: (i, 0)),
        ],
        out_shape=[
            jax.ShapeDtypeStruct((_N8, 128), jnp.float32),
            jax.ShapeDtypeStruct((_N8, 128), jnp.float32),
        ],
    )(agg2_v, y2_pk, deg_v, R, b2_p, S)


def kernel(x, edge_index, W1, b1, W2, b2):
    row3 = edge_index[0].astype(jnp.int32).reshape(-1, _MACRO)
    col3 = edge_index[1].astype(jnp.int32).reshape(-1, _MACRO)
    ones1 = jnp.ones((_MACRO,), jnp.float32)
    zeros1 = jnp.zeros((_NA // _NTILES,), jnp.float32)
    zeros_n16 = jnp.zeros((_NA // _NTILES, 16), jnp.float32)

    eye8 = jnp.eye(8, dtype=jnp.float32)
    W1big = jnp.kron(eye8, W1).reshape(8, 128, 128)
    W2big = jnp.stack([jnp.kron(eye8, W2[:, :16]),
                       jnp.kron(eye8, W2[:, 16:])])
    S = jnp.kron(eye8, jnp.ones((16, 16), jnp.float32))
    b1_p = jnp.tile(b1, 8)[None]
    b2_p = jnp.stack([jnp.tile(b2[:16], 8)[None],
                      jnp.tile(b2[16:], 8)[None]])
    R = jnp.repeat(jnp.eye(8, dtype=jnp.float32), 16, axis=1)

    x3 = x.reshape(-1, 8, 128)

    deg = _deg_kernel(row3, ones1, zeros1)
    deg_v = deg.reshape(2, _N8, 8)
    h1_pk = _tc1a(x3, W1big)
    y1_pk = _tc1b(h1_pk, deg_v, R)
    agg1 = _agg_edge_split(y1_pk.reshape(_NA, 16), row3, col3, zeros_n16)
    y2_pk = _tc2(agg1.reshape(2, _N8, 128), y1_pk, deg_v, R, b1_p, W2big)
    agg2 = _agg_feat_split(y2_pk.reshape(2, _NA, 16), row3, col3, zeros_n16)
    oa, ob = _tc3(agg2.reshape(2, _N8, 128), y2_pk, deg_v, R, b2_p, S)
    return jnp.concatenate([oa.reshape(_NA, 16)[:_N],
                            ob.reshape(_NA, 16)[:_N]], axis=1)


# speedup vs baseline: 1.0460x; 1.0287x over previous
import functools

import jax
import jax.numpy as jnp
from jax import lax
from jax.experimental import pallas as pl
from jax.experimental.pallas import tpu as pltpu
from jax.experimental.pallas import tpu_sc as plsc

_N = 100000
_E = 3200000
_NP = 102400
_NA = 100352
_CH = 80
_K = 10
_MACRO = _CH * _K
_NTILES = 16
_NW = 32

_B = 2048
_G = (_N + _B - 1) // _B


def _sc_mesh():
    return plsc.VectorSubcoreMesh(core_axis_name="c", subcore_axis_name="s")


@functools.partial(
    pl.kernel,
    out_type=jax.ShapeDtypeStruct((2, _NA), jnp.float32),
    mesh=_sc_mesh(),
    compiler_params=pltpu.CompilerParams(use_tc_tiling_on_sc=False),
    scratch_types=[
        pltpu.VMEM_SHARED((_NA,), jnp.float32),
        pltpu.VMEM((3, _MACRO), jnp.int32),
        pltpu.VMEM((_MACRO,), jnp.float32),
        pltpu.SemaphoreType.DMA,
        pltpu.SemaphoreType.DMA,
    ],
)
def _deg_kernel(row3_hbm, ones_hbm, zeros_hbm, out_hbm, acc, ridx, ones,
                isem, ssem):
    c = lax.axis_index("c")
    s = lax.axis_index("s")
    wid = c * _NTILES + s
    pltpu.sync_copy(ones_hbm, ones)
    zr = _NA // _NTILES
    pltpu.sync_copy(zeros_hbm, acc.at[pl.ds(s * zr, zr)])
    plsc.subcore_barrier()

    nm = _E // _MACRO // _NW
    base_m = wid * nm

    pltpu.async_copy(row3_hbm.at[base_m], ridx.at[0], isem)
    pltpu.async_copy(row3_hbm.at[base_m + 1], ridx.at[1], isem)

    @pl.loop(0, nm)
    def _loop(t):
        p = lax.rem(t, 3)
        p1 = lax.rem(t + 2, 3)
        pltpu.make_async_copy(row3_hbm.at[base_m], ridx.at[p], isem).wait()

        @pl.when(t >= 1)
        def _():
            pltpu.make_async_copy(ones, acc.at[ridx.at[p]], ssem).wait()

        @pl.when(t + 2 < nm)
        def _():
            pltpu.async_copy(row3_hbm.at[base_m + t + 2], ridx.at[p1], isem)

        pltpu.async_copy(ones, acc.at[ridx.at[p]], ssem, add=True)

    pltpu.make_async_copy(ones, acc.at[ridx.at[0]], ssem).wait()
    plsc.subcore_barrier()
    pltpu.sync_copy(acc.at[pl.ds(s * zr, zr)], out_hbm.at[c, pl.ds(s * zr, zr)])


def _make_agg(edge_split):
    nm = (_E // _MACRO) // (_NW if edge_split else _NTILES)
    table_shape = (_NA, 16) if edge_split else (2, _NA, 16)

    @functools.partial(
        pl.kernel,
        out_type=jax.ShapeDtypeStruct((2, _NA, 16), jnp.float32),
        mesh=_sc_mesh(),
        compiler_params=pltpu.CompilerParams(use_tc_tiling_on_sc=False),
        scratch_types=[
            pltpu.VMEM_SHARED((_NA, 16), jnp.float32),
            pltpu.VMEM((3, _MACRO), jnp.int32),
            pltpu.VMEM((3, _MACRO), jnp.int32),
            pltpu.VMEM((2, _MACRO, 16), jnp.float32),
            pltpu.SemaphoreType.DMA,
            pltpu.SemaphoreType.DMA,
            pltpu.SemaphoreType.DMA,
        ],
    )
    def agg(table_hbm, row3_hbm, col3_hbm, zeros_hbm, out_hbm,
            acc, ridx, cidx, rows, isem, gsem, ssem):
        c = lax.axis_index("c")
        s = lax.axis_index("s")
        base_m = (c * _NTILES + s if edge_split else s) * nm
        tbl = table_hbm if edge_split else table_hbm.at[c]

        zr = _NA // _NTILES
        pltpu.sync_copy(zeros_hbm, acc.at[pl.ds(s * zr, zr)])
        plsc.subcore_barrier()

        pltpu.async_copy(row3_hbm.at[base_m], ridx.at[0], isem)
        pltpu.async_copy(col3_hbm.at[base_m], cidx.at[0], isem)
        pltpu.async_copy(row3_hbm.at[base_m + 1], ridx.at[1], isem)
        pltpu.async_copy(col3_hbm.at[base_m + 1], cidx.at[1], isem)

        @pl.loop(0, nm)
        def _loop(t):
            p = lax.rem(t, 3)
            p1 = lax.rem(t + 2, 3)
            p2 = lax.rem(t, 2)
            pltpu.make_async_copy(row3_hbm.at[base_m], ridx.at[p], isem).wait()
            pltpu.make_async_copy(col3_hbm.at[base_m], cidx.at[p], isem).wait()
            gc = pltpu.async_copy(tbl.at[ridx.at[p]], rows.at[p2], gsem)

            @pl.when(t >= 1)
            def _():
                pltpu.make_async_copy(zeros_hbm, rows.at[0], ssem).wait()

            @pl.when(t + 2 < nm)
            def _():
                pltpu.async_copy(row3_hbm.at[base_m + t + 2], ridx.at[p1], isem)
                pltpu.async_copy(col3_hbm.at[base_m + t + 2], cidx.at[p1], isem)

            gc.wait()
            pltpu.async_copy(rows.at[p2], acc.at[cidx.at[p]], ssem, add=True)

        pltpu.make_async_copy(zeros_hbm, rows.at[0], ssem).wait()

        plsc.subcore_barrier()
        pltpu.sync_copy(acc.at[pl.ds(s * zr, zr)],
                        out_hbm.at[c, pl.ds(s * zr, zr)])

    return agg


_agg_edge_split = _make_agg(True)
_agg_feat_split = _make_agg(False)


_B8 = _B // 8
_N8 = _NA // 8


def _dis_pk(deg_ref, r_ref):
    d8 = deg_ref[0] + deg_ref[1] + 1.0
    return jnp.dot(lax.rsqrt(d8), r_ref[...],
                   preferred_element_type=jnp.float32)


def _tc1a_body(x_ref, w_ref, h1_ref):
    h = jnp.dot(x_ref[:, 0, :], w_ref[0],
                preferred_element_type=jnp.float32)
    for i in range(1, 8):
        h += jnp.dot(x_ref[:, i, :], w_ref[i],
                     preferred_element_type=jnp.float32)
    h1_ref[...] = h


def _tc1a(x3, W1big):
    return pl.pallas_call(
        _tc1a_body,
        grid=(_G,),
        in_specs=[
            pl.BlockSpec((_B8, 8, 128), lambda i: (i, 0, 0)),
            pl.BlockSpec((8, 128, 128), lambda i: (0, 0, 0)),
        ],
        out_specs=pl.BlockSpec((_B8, 128), lambda i: (i, 0)),
        out_shape=jax.ShapeDtypeStruct((_N8, 128), jnp.float32),
    )(x3, W1big)


def _tc1b_body(h1_ref, deg_ref, r_ref, y1_ref):
    y1_ref[...] = h1_ref[...] * _dis_pk(deg_ref, r_ref)


def _tc1b(h1_pk, deg_v, R):
    return pl.pallas_call(
        _tc1b_body,
        grid=(_G,),
        in_specs=[
            pl.BlockSpec((_B8, 128), lambda i: (i, 0)),
            pl.BlockSpec((2, _B8, 8), lambda i: (0, i, 0)),
            pl.BlockSpec((8, 128), lambda i: (0, 0)),
        ],
        out_specs=pl.BlockSpec((_B8, 128), lambda i: (i, 0)),
        out_shape=jax.ShapeDtypeStruct((_N8, 128), jnp.float32),
    )(h1_pk, deg_v, R)


def _tc2_body(ag_ref, y1_ref, deg_ref, r_ref, b1_ref, w2_ref, y2_ref):
    dis = _dis_pk(deg_ref, r_ref)
    h = jnp.maximum(dis * (ag_ref[0] + ag_ref[1] + y1_ref[...]) + b1_ref[...],
                    0.0)
    y2_ref[0] = dis * jnp.dot(h, w2_ref[0],
                              preferred_element_type=jnp.float32)
    y2_ref[1] = dis * jnp.dot(h, w2_ref[1],
                              preferred_element_type=jnp.float32)


def _tc2(agg1_v, y1_pk, deg_v, R, b1_p, W2big):
    return pl.pallas_call(
        _tc2_body,
        grid=(_G,),
        in_specs=[
            pl.BlockSpec((2, _B8, 128), lambda i: (0, i, 0)),
            pl.BlockSpec((_B8, 128), lambda i: (i, 0)),
            pl.BlockSpec((2, _B8, 8), lambda i: (0, i, 0)),
            pl.BlockSpec((8, 128), lambda i: (0, 0)),
            pl.BlockSpec((1, 128), lambda i: (0, 0)),
            pl.BlockSpec((2, 128, 128), lambda i: (0, 0, 0)),
        ],
        out_specs=pl.BlockSpec((2, _B8, 128), lambda i: (0, i, 0)),
        out_shape=jax.ShapeDtypeStruct((2, _N8, 128), jnp.float32),
    )(agg1_v, y1_pk, deg_v, R, b1_p, W2big)


def _tc3_body(ag_ref, y2_ref, deg_ref, r_ref, b2_ref, s_ref, oa_ref, ob_ref):
    dis = _dis_pk(deg_ref, r_ref)
    za = dis * (ag_ref[0] + y2_ref[0]) + b2_ref[0]
    zb = dis * (ag_ref[1] + y2_ref[1]) + b2_ref[1]
    m8 = jnp.max(jnp.maximum(za, zb), axis=1, keepdims=True)
    ea = jnp.exp(za - m8)
    eb = jnp.exp(zb - m8)
    se = jnp.dot(ea + eb, s_ref[...], preferred_element_type=jnp.float32)
    lse = m8 + jnp.log(se)
    oa_ref[...] = za - lse
    ob_ref[...] = zb - lse


def _tc3(agg2_v, y2_pk, deg_v, R, b2_p, S):
    return pl.pallas_call(
        _tc3_body,
        grid=(_G,),
        in_specs=[
            pl.BlockSpec((2, _B8, 128), lambda i: (0, i, 0)),
            pl.BlockSpec((2, _B8, 128), lambda i: (0, i, 0)),
            pl.BlockSpec((2, _B8, 8), lambda i: (0, i, 0)),
            pl.BlockSpec((8, 128), lambda i: (0, 0)),
            pl.BlockSpec((2, 1, 128), lambda i: (0, 0, 0)),
            pl.BlockSpec((128, 128), lambda i: (0, 0)),
        ],
        out_specs=[
            pl.BlockSpec((_B8, 128), lambda i: (i, 0)),
            pl.BlockSpec((_B8, 128), lambda i: (i, 0)),
        ],
        out_shape=[
            jax.ShapeDtypeStruct((_N8, 128), jnp.float32),
            jax.ShapeDtypeStruct((_N8, 128), jnp.float32),
        ],
    )(agg2_v, y2_pk, deg_v, R, b2_p, S)


def kernel(x, edge_index, W1, b1, W2, b2):
    row3 = edge_index[0].astype(jnp.int32).reshape(-1, _MACRO)
    col3 = edge_index[1].astype(jnp.int32).reshape(-1, _MACRO)
    ones1 = jnp.ones((_MACRO,), jnp.float32)
    zeros1 = jnp.zeros((_NA // _NTILES,), jnp.float32)
    zeros_n16 = jnp.zeros((_NA // _NTILES, 16), jnp.float32)

    eye8 = jnp.eye(8, dtype=jnp.float32)
    W1big = jnp.kron(eye8, W1).reshape(8, 128, 128)
    W2big = jnp.stack([jnp.kron(eye8, W2[:, :16]),
                       jnp.kron(eye8, W2[:, 16:])])
    S = jnp.kron(eye8, jnp.ones((16, 16), jnp.float32))
    b1_p = jnp.tile(b1, 8)[None]
    b2_p = jnp.stack([jnp.tile(b2[:16], 8)[None],
                      jnp.tile(b2[16:], 8)[None]])
    R = jnp.repeat(jnp.eye(8, dtype=jnp.float32), 16, axis=1)

    x3 = x.reshape(-1, 8, 128)

    deg = _deg_kernel(row3, ones1, zeros1)
    deg_v = deg.reshape(2, _N8, 8)
    h1_pk = _tc1a(x3, W1big)
    y1_pk = _tc1b(h1_pk, deg_v, R)
    agg1 = _agg_edge_split(y1_pk.reshape(_NA, 16), row3, col3, zeros_n16)
    y2_pk = _tc2(agg1.reshape(2, _N8, 128), y1_pk, deg_v, R, b1_p, W2big)
    agg2 = _agg_feat_split(y2_pk.reshape(2, _NA, 16), row3, col3, zeros_n16)
    oa, ob = _tc3(agg2.reshape(2, _N8, 128), y2_pk, deg_v, R, b2_p, S)
    return jnp.concatenate([oa.reshape(_NA, 16)[:_N],
                            ob.reshape(_NA, 16)[:_N]], axis=1)


# speedup vs baseline: 1.3280x; 1.2696x over previous
import functools

import jax
import jax.numpy as jnp
from jax import lax
from jax.experimental import pallas as pl
from jax.experimental.pallas import tpu as pltpu
from jax.experimental.pallas import tpu_sc as plsc

_N = 100000
_E = 3200000
_NP = 102400
_NA = 100352
_CH = 80
_K = 10
_MACRO = _CH * _K
_NTILES = 16
_NW = 32

_B = 2048
_G = (_N + _B - 1) // _B


def _sc_mesh():
    return plsc.VectorSubcoreMesh(core_axis_name="c", subcore_axis_name="s")


@functools.partial(
    pl.kernel,
    out_type=jax.ShapeDtypeStruct((2, _NA), jnp.float32),
    mesh=_sc_mesh(),
    compiler_params=pltpu.CompilerParams(use_tc_tiling_on_sc=False),
    scratch_types=[
        pltpu.VMEM_SHARED((_NA,), jnp.float32),
        pltpu.VMEM((3, _MACRO), jnp.int32),
        pltpu.VMEM((_MACRO,), jnp.float32),
        pltpu.SemaphoreType.DMA,
        pltpu.SemaphoreType.DMA,
    ],
)
def _deg_kernel(row3_hbm, ones_hbm, zeros_hbm, out_hbm, acc, ridx, ones,
                isem, ssem):
    c = lax.axis_index("c")
    s = lax.axis_index("s")
    wid = c * _NTILES + s
    pltpu.sync_copy(ones_hbm, ones)
    zr = _NA // _NTILES
    pltpu.sync_copy(zeros_hbm, acc.at[pl.ds(s * zr, zr)])
    plsc.subcore_barrier()

    nm = _E // _MACRO // _NW
    base_m = wid * nm

    pltpu.async_copy(row3_hbm.at[base_m], ridx.at[0], isem)
    pltpu.async_copy(row3_hbm.at[base_m + 1], ridx.at[1], isem)

    @pl.loop(0, nm)
    def _loop(t):
        p = lax.rem(t, 3)
        p1 = lax.rem(t + 2, 3)
        pltpu.make_async_copy(row3_hbm.at[base_m], ridx.at[p], isem).wait()

        @pl.when(t >= 1)
        def _():
            pltpu.make_async_copy(ones, acc.at[ridx.at[p]], ssem).wait()

        @pl.when(t + 2 < nm)
        def _():
            pltpu.async_copy(row3_hbm.at[base_m + t + 2], ridx.at[p1], isem)

        pltpu.async_copy(ones, acc.at[ridx.at[p]], ssem, add=True)

    pltpu.make_async_copy(ones, acc.at[ridx.at[0]], ssem).wait()
    plsc.subcore_barrier()
    pltpu.sync_copy(acc.at[pl.ds(s * zr, zr)], out_hbm.at[c, pl.ds(s * zr, zr)])


def _make_agg(edge_split):
    nm = (_E // _MACRO) // (_NW if edge_split else _NTILES)
    table_shape = (_NA, 16) if edge_split else (2, _NA, 16)

    @functools.partial(
        pl.kernel,
        out_type=jax.ShapeDtypeStruct((2, _NA, 16), jnp.float32),
        mesh=_sc_mesh(),
        compiler_params=pltpu.CompilerParams(use_tc_tiling_on_sc=False),
        scratch_types=[
            pltpu.VMEM_SHARED((_NA, 16), jnp.float32),
            pltpu.VMEM((3, _MACRO), jnp.int32),
            pltpu.VMEM((3, _MACRO), jnp.int32),
            pltpu.VMEM((2, _MACRO, 16), jnp.float32),
            pltpu.SemaphoreType.DMA,
            pltpu.SemaphoreType.DMA,
            pltpu.SemaphoreType.DMA,
        ],
    )
    def agg(table_hbm, row3_hbm, col3_hbm, zeros_hbm, out_hbm,
            acc, ridx, cidx, rows, isem, gsem, ssem):
        c = lax.axis_index("c")
        s = lax.axis_index("s")
        base_m = (c * _NTILES + s if edge_split else s) * nm
        tbl = table_hbm if edge_split else table_hbm.at[c]

        zr = _NA // _NTILES
        pltpu.sync_copy(zeros_hbm, acc.at[pl.ds(s * zr, zr)])
        plsc.subcore_barrier()

        pltpu.async_copy(row3_hbm.at[base_m], ridx.at[0], isem)
        pltpu.async_copy(col3_hbm.at[base_m], cidx.at[0], isem)
        pltpu.async_copy(row3_hbm.at[base_m + 1], ridx.at[1], isem)
        pltpu.async_copy(col3_hbm.at[base_m + 1], cidx.at[1], isem)

        @pl.loop(0, nm)
        def _loop(t):
            p = lax.rem(t, 3)
            p1 = lax.rem(t + 2, 3)
            p2 = lax.rem(t, 2)
            pltpu.make_async_copy(row3_hbm.at[base_m], ridx.at[p], isem).wait()
            pltpu.make_async_copy(col3_hbm.at[base_m], cidx.at[p], isem).wait()
            gc = pltpu.async_copy(tbl.at[ridx.at[p]], rows.at[p2], gsem)

            @pl.when(t >= 1)
            def _():
                pltpu.make_async_copy(zeros_hbm, rows.at[0], ssem).wait()

            @pl.when(t + 2 < nm)
            def _():
                pltpu.async_copy(row3_hbm.at[base_m + t + 2], ridx.at[p1], isem)
                pltpu.async_copy(col3_hbm.at[base_m + t + 2], cidx.at[p1], isem)

            gc.wait()
            pltpu.async_copy(rows.at[p2], acc.at[cidx.at[p]], ssem, add=True)

        pltpu.make_async_copy(zeros_hbm, rows.at[0], ssem).wait()

        plsc.subcore_barrier()
        pltpu.sync_copy(acc.at[pl.ds(s * zr, zr)],
                        out_hbm.at[c, pl.ds(s * zr, zr)])

    return agg


_agg_edge_split = _make_agg(True)


_B8 = _B // 8
_N8 = _NA // 8


def _dis_pk(deg_ref, r_ref):
    d8 = deg_ref[0] + deg_ref[1] + 1.0
    return jnp.dot(lax.rsqrt(d8), r_ref[...],
                   preferred_element_type=jnp.float32)


def _tc1a_body(x_ref, w_ref, h1_ref):
    h = jnp.dot(x_ref[:, 0, :], w_ref[0],
                preferred_element_type=jnp.float32)
    for i in range(1, 8):
        h += jnp.dot(x_ref[:, i, :], w_ref[i],
                     preferred_element_type=jnp.float32)
    h1_ref[...] = h


def _tc1a(x3, W1big):
    return pl.pallas_call(
        _tc1a_body,
        grid=(_G,),
        in_specs=[
            pl.BlockSpec((_B8, 8, 128), lambda i: (i, 0, 0)),
            pl.BlockSpec((8, 128, 128), lambda i: (0, 0, 0)),
        ],
        out_specs=pl.BlockSpec((_B8, 128), lambda i: (i, 0)),
        out_shape=jax.ShapeDtypeStruct((_N8, 128), jnp.float32),
    )(x3, W1big)


def _tc1b_body(h1_ref, deg_ref, r_ref, y1_ref):
    y1_ref[...] = h1_ref[...] * _dis_pk(deg_ref, r_ref)


def _tc1b(h1_pk, deg_v, R):
    return pl.pallas_call(
        _tc1b_body,
        grid=(_G,),
        in_specs=[
            pl.BlockSpec((_B8, 128), lambda i: (i, 0)),
            pl.BlockSpec((2, _B8, 8), lambda i: (0, i, 0)),
            pl.BlockSpec((8, 128), lambda i: (0, 0)),
        ],
        out_specs=pl.BlockSpec((_B8, 128), lambda i: (i, 0)),
        out_shape=jax.ShapeDtypeStruct((_N8, 128), jnp.float32),
    )(h1_pk, deg_v, R)


def _tc2_body(ag_ref, y1_ref, deg_ref, r_ref, b1_ref, g_ref):
    dis = _dis_pk(deg_ref, r_ref)
    h = jnp.maximum(dis * (ag_ref[0] + ag_ref[1] + y1_ref[...]) + b1_ref[...],
                    0.0)
    g_ref[...] = dis * h


def _tc2(agg1_v, y1_pk, deg_v, R, b1_p):
    return pl.pallas_call(
        _tc2_body,
        grid=(_G,),
        in_specs=[
            pl.BlockSpec((2, _B8, 128), lambda i: (0, i, 0)),
            pl.BlockSpec((_B8, 128), lambda i: (i, 0)),
            pl.BlockSpec((2, _B8, 8), lambda i: (0, i, 0)),
            pl.BlockSpec((8, 128), lambda i: (0, 0)),
            pl.BlockSpec((1, 128), lambda i: (0, 0)),
        ],
        out_specs=pl.BlockSpec((_B8, 128), lambda i: (i, 0)),
        out_shape=jax.ShapeDtypeStruct((_N8, 128), jnp.float32),
    )(agg1_v, y1_pk, deg_v, R, b1_p)


def _tc3_body(ag_ref, g_ref, deg_ref, r_ref, w2_ref, b2_ref, s_ref,
              oa_ref, ob_ref):
    dis = _dis_pk(deg_ref, r_ref)
    s2 = dis * (ag_ref[0] + ag_ref[1] + g_ref[...])
    za = jnp.dot(s2, w2_ref[0], preferred_element_type=jnp.float32) + b2_ref[0]
    zb = jnp.dot(s2, w2_ref[1], preferred_element_type=jnp.float32) + b2_ref[1]
    m8 = jnp.max(jnp.maximum(za, zb), axis=1, keepdims=True)
    ea = jnp.exp(za - m8)
    eb = jnp.exp(zb - m8)
    se = jnp.dot(ea + eb, s_ref[...], preferred_element_type=jnp.float32)
    lse = m8 + jnp.log(se)
    oa_ref[...] = za - lse
    ob_ref[...] = zb - lse


def _tc3(agg2_v, g_pk, deg_v, R, W2big, b2_p, S):
    return pl.pallas_call(
        _tc3_body,
        grid=(_G,),
        in_specs=[
            pl.BlockSpec((2, _B8, 128), lambda i: (0, i, 0)),
            pl.BlockSpec((_B8, 128), lambda i: (i, 0)),
            pl.BlockSpec((2, _B8, 8), lambda i: (0, i, 0)),
            pl.BlockSpec((8, 128), lambda i: (0, 0)),
            pl.BlockSpec((2, 128, 128), lambda i: (0, 0, 0)),
            pl.BlockSpec((2, 1, 128), lambda i: (0, 0, 0)),
            pl.BlockSpec((128, 128), lambda i: (0, 0)),
        ],
        out_specs=[
            pl.BlockSpec((_B8, 128), lambda i: (i, 0)),
            pl.BlockSpec((_B8, 128), lambda i: (i, 0)),
        ],
        out_shape=[
            jax.ShapeDtypeStruct((_N8, 128), jnp.float32),
            jax.ShapeDtypeStruct((_N8, 128), jnp.float32),
        ],
    )(agg2_v, g_pk, deg_v, R, W2big, b2_p, S)


def kernel(x, edge_index, W1, b1, W2, b2):
    row3 = edge_index[0].astype(jnp.int32).reshape(-1, _MACRO)
    col3 = edge_index[1].astype(jnp.int32).reshape(-1, _MACRO)
    ones1 = jnp.ones((_MACRO,), jnp.float32)
    zeros1 = jnp.zeros((_NA // _NTILES,), jnp.float32)
    zeros_n16 = jnp.zeros((_NA // _NTILES, 16), jnp.float32)

    eye8 = jnp.eye(8, dtype=jnp.float32)
    W1big = jnp.kron(eye8, W1).reshape(8, 128, 128)
    W2big = jnp.stack([jnp.kron(eye8, W2[:, :16]),
                       jnp.kron(eye8, W2[:, 16:])])
    S = jnp.kron(eye8, jnp.ones((16, 16), jnp.float32))
    b1_p = jnp.tile(b1, 8)[None]
    b2_p = jnp.stack([jnp.tile(b2[:16], 8)[None],
                      jnp.tile(b2[16:], 8)[None]])
    R = jnp.repeat(jnp.eye(8, dtype=jnp.float32), 16, axis=1)

    x3 = x.reshape(-1, 8, 128)

    deg = _deg_kernel(row3, ones1, zeros1)
    deg_v = deg.reshape(2, _N8, 8)
    h1_pk = _tc1a(x3, W1big)
    y1_pk = _tc1b(h1_pk, deg_v, R)
    agg1 = _agg_edge_split(y1_pk.reshape(_NA, 16), row3, col3, zeros_n16)
    g_pk = _tc2(agg1.reshape(2, _N8, 128), y1_pk, deg_v, R, b1_p)
    agg2 = _agg_edge_split(g_pk.reshape(_NA, 16), row3, col3, zeros_n16)
    oa, ob = _tc3(agg2.reshape(2, _N8, 128), g_pk, deg_v, R, W2big, b2_p, S)
    return jnp.concatenate([oa.reshape(_NA, 16)[:_N],
                            ob.reshape(_NA, 16)[:_N]], axis=1)


# speedup vs baseline: 1.3285x; 1.0003x over previous
import functools

import jax
import jax.numpy as jnp
from jax import lax
from jax.experimental import pallas as pl
from jax.experimental.pallas import tpu as pltpu
from jax.experimental.pallas import tpu_sc as plsc

_N = 100000
_E = 3200000
_NP = 102400
_NA = 100352
_CH = 80
_K = 10
_MACRO = _CH * _K
_DEGM = 2000
_NTILES = 16
_NW = 32

_B = 2048
_G = (_N + _B - 1) // _B


def _sc_mesh():
    return plsc.VectorSubcoreMesh(core_axis_name="c", subcore_axis_name="s")


@functools.partial(
    pl.kernel,
    out_type=jax.ShapeDtypeStruct((2, _NA), jnp.float32),
    mesh=_sc_mesh(),
    compiler_params=pltpu.CompilerParams(use_tc_tiling_on_sc=False),
    scratch_types=[
        pltpu.VMEM_SHARED((_NA,), jnp.float32),
        pltpu.VMEM((3, _DEGM), jnp.int32),
        pltpu.VMEM((_DEGM,), jnp.float32),
        pltpu.SemaphoreType.DMA,
        pltpu.SemaphoreType.DMA,
    ],
)
def _deg_kernel(row3_hbm, ones_hbm, zeros_hbm, out_hbm, acc, ridx, ones,
                isem, ssem):
    c = lax.axis_index("c")
    s = lax.axis_index("s")
    wid = c * _NTILES + s
    pltpu.sync_copy(ones_hbm, ones)
    zr = _NA // _NTILES
    pltpu.sync_copy(zeros_hbm, acc.at[pl.ds(s * zr, zr)])
    plsc.subcore_barrier()

    nm = _E // _DEGM // _NW
    base_m = wid * nm

    pltpu.async_copy(row3_hbm.at[base_m], ridx.at[0], isem)
    pltpu.async_copy(row3_hbm.at[base_m + 1], ridx.at[1], isem)

    @pl.loop(0, nm)
    def _loop(t):
        p = lax.rem(t, 3)
        p1 = lax.rem(t + 2, 3)
        pltpu.make_async_copy(row3_hbm.at[base_m], ridx.at[p], isem).wait()

        @pl.when(t >= 1)
        def _():
            pltpu.make_async_copy(ones, acc.at[ridx.at[p]], ssem).wait()

        @pl.when(t + 2 < nm)
        def _():
            pltpu.async_copy(row3_hbm.at[base_m + t + 2], ridx.at[p1], isem)

        pltpu.async_copy(ones, acc.at[ridx.at[p]], ssem, add=True)

    pltpu.make_async_copy(ones, acc.at[ridx.at[0]], ssem).wait()
    plsc.subcore_barrier()
    pltpu.sync_copy(acc.at[pl.ds(s * zr, zr)], out_hbm.at[c, pl.ds(s * zr, zr)])


def _make_agg(edge_split):
    nm = (_E // _MACRO) // (_NW if edge_split else _NTILES)
    table_shape = (_NA, 16) if edge_split else (2, _NA, 16)

    @functools.partial(
        pl.kernel,
        out_type=jax.ShapeDtypeStruct((2, _NA, 16), jnp.float32),
        mesh=_sc_mesh(),
        compiler_params=pltpu.CompilerParams(use_tc_tiling_on_sc=False),
        scratch_types=[
            pltpu.VMEM_SHARED((_NA, 16), jnp.float32),
            pltpu.VMEM((3, _MACRO), jnp.int32),
            pltpu.VMEM((3, _MACRO), jnp.int32),
            pltpu.VMEM((2, _MACRO, 16), jnp.float32),
            pltpu.SemaphoreType.DMA,
            pltpu.SemaphoreType.DMA,
            pltpu.SemaphoreType.DMA,
        ],
    )
    def agg(table_hbm, row3_hbm, col3_hbm, zeros_hbm, out_hbm,
            acc, ridx, cidx, rows, isem, gsem, ssem):
        c = lax.axis_index("c")
        s = lax.axis_index("s")
        base_m = (c * _NTILES + s if edge_split else s) * nm
        tbl = table_hbm if edge_split else table_hbm.at[c]

        zr = _NA // _NTILES
        pltpu.sync_copy(zeros_hbm, acc.at[pl.ds(s * zr, zr)])
        plsc.subcore_barrier()

        pltpu.async_copy(row3_hbm.at[base_m], ridx.at[0], isem)
        pltpu.async_copy(col3_hbm.at[base_m], cidx.at[0], isem)
        pltpu.async_copy(row3_hbm.at[base_m + 1], ridx.at[1], isem)
        pltpu.async_copy(col3_hbm.at[base_m + 1], cidx.at[1], isem)

        @pl.loop(0, nm)
        def _loop(t):
            p = lax.rem(t, 3)
            p1 = lax.rem(t + 2, 3)
            p2 = lax.rem(t, 2)
            pltpu.make_async_copy(row3_hbm.at[base_m], ridx.at[p], isem).wait()
            pltpu.make_async_copy(col3_hbm.at[base_m], cidx.at[p], isem).wait()
            gc = pltpu.async_copy(tbl.at[ridx.at[p]], rows.at[p2], gsem)

            @pl.when(t >= 1)
            def _():
                pltpu.make_async_copy(zeros_hbm, rows.at[0], ssem).wait()

            @pl.when(t + 2 < nm)
            def _():
                pltpu.async_copy(row3_hbm.at[base_m + t + 2], ridx.at[p1], isem)
                pltpu.async_copy(col3_hbm.at[base_m + t + 2], cidx.at[p1], isem)

            gc.wait()
            pltpu.async_copy(rows.at[p2], acc.at[cidx.at[p]], ssem, add=True)

        pltpu.make_async_copy(zeros_hbm, rows.at[0], ssem).wait()

        plsc.subcore_barrier()
        pltpu.sync_copy(acc.at[pl.ds(s * zr, zr)],
                        out_hbm.at[c, pl.ds(s * zr, zr)])

    return agg


_agg_edge_split = _make_agg(True)


_B8 = _B // 8
_N8 = _NA // 8


def _dis_pk(deg_ref, r_ref):
    d8 = deg_ref[0] + deg_ref[1] + 1.0
    return jnp.dot(lax.rsqrt(d8), r_ref[...],
                   preferred_element_type=jnp.float32)


def _tc1a_body(x_ref, w_ref, h1_ref):
    h = jnp.dot(x_ref[:, 0, :], w_ref[0],
                preferred_element_type=jnp.float32)
    for i in range(1, 8):
        h += jnp.dot(x_ref[:, i, :], w_ref[i],
                     preferred_element_type=jnp.float32)
    h1_ref[...] = h


def _tc1a(x3, W1big):
    return pl.pallas_call(
        _tc1a_body,
        grid=(_G,),
        in_specs=[
            pl.BlockSpec((_B8, 8, 128), lambda i: (i, 0, 0)),
            pl.BlockSpec((8, 128, 128), lambda i: (0, 0, 0)),
        ],
        out_specs=pl.BlockSpec((_B8, 128), lambda i: (i, 0)),
        out_shape=jax.ShapeDtypeStruct((_N8, 128), jnp.float32),
    )(x3, W1big)


def _tc1b_body(h1_ref, deg_ref, r_ref, y1_ref):
    y1_ref[...] = h1_ref[...] * _dis_pk(deg_ref, r_ref)


def _tc1b(h1_pk, deg_v, R):
    return pl.pallas_call(
        _tc1b_body,
        grid=(_G,),
        in_specs=[
            pl.BlockSpec((_B8, 128), lambda i: (i, 0)),
            pl.BlockSpec((2, _B8, 8), lambda i: (0, i, 0)),
            pl.BlockSpec((8, 128), lambda i: (0, 0)),
        ],
        out_specs=pl.BlockSpec((_B8, 128), lambda i: (i, 0)),
        out_shape=jax.ShapeDtypeStruct((_N8, 128), jnp.float32),
    )(h1_pk, deg_v, R)


def _tc2_body(ag_ref, y1_ref, deg_ref, r_ref, b1_ref, g_ref):
    dis = _dis_pk(deg_ref, r_ref)
    h = jnp.maximum(dis * (ag_ref[0] + ag_ref[1] + y1_ref[...]) + b1_ref[...],
                    0.0)
    g_ref[...] = dis * h


def _tc2(agg1_v, y1_pk, deg_v, R, b1_p):
    return pl.pallas_call(
        _tc2_body,
        grid=(_G,),
        in_specs=[
            pl.BlockSpec((2, _B8, 128), lambda i: (0, i, 0)),
            pl.BlockSpec((_B8, 128), lambda i: (i, 0)),
            pl.BlockSpec((2, _B8, 8), lambda i: (0, i, 0)),
            pl.BlockSpec((8, 128), lambda i: (0, 0)),
            pl.BlockSpec((1, 128), lambda i: (0, 0)),
        ],
        out_specs=pl.BlockSpec((_B8, 128), lambda i: (i, 0)),
        out_shape=jax.ShapeDtypeStruct((_N8, 128), jnp.float32),
    )(agg1_v, y1_pk, deg_v, R, b1_p)


def _tc3_body(ag_ref, g_ref, deg_ref, r_ref, w2_ref, b2_ref, s_ref,
              oa_ref, ob_ref):
    dis = _dis_pk(deg_ref, r_ref)
    s2 = dis * (ag_ref[0] + ag_ref[1] + g_ref[...])
    za = jnp.dot(s2, w2_ref[0], preferred_element_type=jnp.float32) + b2_ref[0]
    zb = jnp.dot(s2, w2_ref[1], preferred_element_type=jnp.float32) + b2_ref[1]
    m8 = jnp.max(jnp.maximum(za, zb), axis=1, keepdims=True)
    ea = jnp.exp(za - m8)
    eb = jnp.exp(zb - m8)
    se = jnp.dot(ea + eb, s_ref[...], preferred_element_type=jnp.float32)
    lse = m8 + jnp.log(se)
    oa_ref[...] = za - lse
    ob_ref[...] = zb - lse


def _tc3(agg2_v, g_pk, deg_v, R, W2big, b2_p, S):
    return pl.pallas_call(
        _tc3_body,
        grid=(_G,),
        in_specs=[
            pl.BlockSpec((2, _B8, 128), lambda i: (0, i, 0)),
            pl.BlockSpec((_B8, 128), lambda i: (i, 0)),
            pl.BlockSpec((2, _B8, 8), lambda i: (0, i, 0)),
            pl.BlockSpec((8, 128), lambda i: (0, 0)),
            pl.BlockSpec((2, 128, 128), lambda i: (0, 0, 0)),
            pl.BlockSpec((2, 1, 128), lambda i: (0, 0, 0)),
            pl.BlockSpec((128, 128), lambda i: (0, 0)),
        ],
        out_specs=[
            pl.BlockSpec((_B8, 128), lambda i: (i, 0)),
            pl.BlockSpec((_B8, 128), lambda i: (i, 0)),
        ],
        out_shape=[
            jax.ShapeDtypeStruct((_N8, 128), jnp.float32),
            jax.ShapeDtypeStruct((_N8, 128), jnp.float32),
        ],
    )(agg2_v, g_pk, deg_v, R, W2big, b2_p, S)


def kernel(x, edge_index, W1, b1, W2, b2):
    row3 = edge_index[0].astype(jnp.int32).reshape(-1, _MACRO)
    col3 = edge_index[1].astype(jnp.int32).reshape(-1, _MACRO)
    ones1 = jnp.ones((_DEGM,), jnp.float32)
    zeros1 = jnp.zeros((_NA // _NTILES,), jnp.float32)
    zeros_n16 = jnp.zeros((_NA // _NTILES, 16), jnp.float32)

    eye8 = jnp.eye(8, dtype=jnp.float32)
    W1big = jnp.kron(eye8, W1).reshape(8, 128, 128)
    W2big = jnp.stack([jnp.kron(eye8, W2[:, :16]),
                       jnp.kron(eye8, W2[:, 16:])])
    S = jnp.kron(eye8, jnp.ones((16, 16), jnp.float32))
    b1_p = jnp.tile(b1, 8)[None]
    b2_p = jnp.stack([jnp.tile(b2[:16], 8)[None],
                      jnp.tile(b2[16:], 8)[None]])
    R = jnp.repeat(jnp.eye(8, dtype=jnp.float32), 16, axis=1)

    x3 = x.reshape(-1, 8, 128)

    rowd = edge_index[0].astype(jnp.int32).reshape(-1, _DEGM)
    deg = _deg_kernel(rowd, ones1, zeros1)
    deg_v = deg.reshape(2, _N8, 8)
    h1_pk = _tc1a(x3, W1big)
    y1_pk = _tc1b(h1_pk, deg_v, R)
    agg1 = _agg_edge_split(y1_pk.reshape(_NA, 16), row3, col3, zeros_n16)
    g_pk = _tc2(agg1.reshape(2, _N8, 128), y1_pk, deg_v, R, b1_p)
    agg2 = _agg_edge_split(g_pk.reshape(_NA, 16), row3, col3, zeros_n16)
    oa, ob = _tc3(agg2.reshape(2, _N8, 128), g_pk, deg_v, R, W2big, b2_p, S)
    return jnp.concatenate([oa.reshape(_NA, 16)[:_N],
                            ob.reshape(_NA, 16)[:_N]], axis=1)
